# baseline (speedup 1.0000x reference)
import functools

import jax
import jax.numpy as jnp
from jax import lax
from jax.experimental import pallas as pl
from jax.experimental.pallas import tpu as pltpu
from jax.experimental.pallas import tpu_sc as plsc

D = 64
DP = 128
N_NUM = 13
N_CAT = 26
VOCAB = 1000
B = 16384
N_TOK = N_NUM + N_CAT
ROW = N_TOK * D
CAT0 = N_NUM * D

NC = 2
NS = 16
NW = NC * NS
BPW = B // NW
BC = 8
NCH = BPW // BC
NPAIR = NCH // 2
L = 16
IPC = BC * N_CAT
GR = 104
NG = IPC // GR
CHW = BC * ROW


def _prep_body(*refs):
    t_refs = refs[:N_CAT]
    cb_ref = refs[N_CAT]
    o_ref = refs[N_CAT + 1]
    rows = jnp.concatenate(
        [t_refs[i][...] + cb_ref[i : i + 1, :] for i in range(N_CAT)], axis=0
    )
    o_ref[...] = jnp.concatenate([rows, jnp.zeros_like(rows)], axis=1)


_prep = pl.pallas_call(
    _prep_body,
    grid=(1,),
    in_specs=[pl.BlockSpec((VOCAB, D), lambda i: (0, 0)) for _ in range(N_CAT)]
    + [pl.BlockSpec((N_CAT, D), lambda i: (0, 0))],
    out_specs=pl.BlockSpec((N_CAT * VOCAB, DP), lambda i: (0, 0)),
    out_shape=jax.ShapeDtypeStruct((N_CAT * VOCAB, DP), jnp.float32),
)


def _tokenizer_kernel(xnum_hbm, xcat_hbm, tbl_hbm, w_hbm, nb_hbm, out_hbm,
                      idx_all, xnum_all, cat_a, cat_b, buf_a, buf_b,
                      w_v, nb_v, lsem, gsem_a, gsem_b, wsem_a, wsem_b):
    wid = lax.axis_index("s") * NC + lax.axis_index("c")
    base = wid * BPW

    cps = [
        pltpu.async_copy(w_hbm, w_v, lsem),
        pltpu.async_copy(nb_hbm, nb_v, lsem),
        pltpu.async_copy(xnum_hbm.at[pl.ds(base * L, BPW * L)], xnum_all, lsem),
        pltpu.async_copy(
            xcat_hbm.at[pl.ds(base * N_CAT, BPW * N_CAT)], idx_all, lsem
        ),
    ]
    for cp in cps:
        cp.wait()

    lane = lax.iota(jnp.int32, L)

    def fix_body(k, _):
        sl = pl.ds(k * L, L)
        p = k * L + lane
        off = VOCAB * lax.rem(p, N_CAT)
        idx_all[sl] = jnp.clip(idx_all[sl], 0, VOCAB - 1) + off
        return 0

    lax.fori_loop(0, BPW * N_CAT // L, fix_body, 0)

    def fire_gathers(c, catbuf, sem):
        for g in range(NG):
            pltpu.async_copy(
                tbl_hbm.at[idx_all.at[pl.ds(c * IPC + g * GR, GR)]],
                catbuf.at[pl.ds(g * GR, GR), :],
                sem,
            )

    def drain_gathers(catbuf, sem):
        for g in range(NG):
            pltpu.make_async_copy(
                tbl_hbm.at[idx_all.at[pl.ds(g * GR, GR)]],
                catbuf.at[pl.ds(g * GR, GR), :],
                sem,
            ).wait()

    def fire_write(c, obuf, sem):
        pltpu.async_copy(obuf, out_hbm.at[pl.ds((base + c * BC) * ROW, CHW)], sem)

    def drain_write(obuf, sem):
        pltpu.make_async_copy(obuf, out_hbm.at[pl.ds(0, CHW)], sem).wait()

    def assemble(c, catbuf, obuf):
        @plsc.parallel_loop(0, IPC, unroll=4)
        def cat_move(m):
            b = m // N_CAT
            i = m - b * N_CAT
            dst = b * ROW + CAT0 + i * D
            vals = [catbuf[m, pl.ds(d4 * L, L)] for d4 in range(D // L)]
            for d4 in range(D // L):
                obuf[pl.ds(dst + d4 * L, L)] = vals[d4]

        xvs = [xnum_all[pl.ds((c * BC + b) * L, L)] for b in range(BC)]
        for j in range(N_NUM):
            for d4 in range(D // L):
                wv = w_v[pl.ds(j * D + d4 * L, L)]
                nbv = nb_v[pl.ds(j * D + d4 * L, L)]
                for b in range(BC):
                    obuf[pl.ds(b * ROW + j * D + d4 * L, L)] = xvs[b][j] * wv + nbv

    fire_gathers(0, cat_a, gsem_a)

    def pair_body(k, _):
        c0 = 2 * k
        c1 = 2 * k + 1

        drain_gathers(cat_a, gsem_a)
        fire_gathers(c1, cat_b, gsem_b)

        @pl.when(k > 0)
        def _():
            drain_write(buf_a, wsem_a)

        assemble(c0, cat_a, buf_a)
        fire_write(c0, buf_a, wsem_a)

        drain_gathers(cat_b, gsem_b)

        @pl.when(k < NPAIR - 1)
        def _():
            fire_gathers(c0 + 2, cat_a, gsem_a)

        @pl.when(k > 0)
        def _():
            drain_write(buf_b, wsem_b)

        assemble(c1, cat_b, buf_b)
        fire_write(c1, buf_b, wsem_b)
        return 0

    lax.fori_loop(0, NPAIR, pair_body, 0)
    drain_write(buf_a, wsem_a)
    drain_write(buf_b, wsem_b)


@functools.partial(
    pl.kernel,
    mesh=plsc.VectorSubcoreMesh(core_axis_name="c", subcore_axis_name="s"),
    out_type=jax.ShapeDtypeStruct((B * ROW,), jnp.float32),
    scratch_types=[
        pltpu.VMEM((N_CAT * BPW,), jnp.int32),
        pltpu.VMEM((BPW * L,), jnp.float32),
        pltpu.VMEM((IPC, DP), jnp.float32),
        pltpu.VMEM((IPC, DP), jnp.float32),
        pltpu.VMEM((CHW,), jnp.float32),
        pltpu.VMEM((CHW,), jnp.float32),
        pltpu.VMEM((N_NUM * D,), jnp.float32),
        pltpu.VMEM((N_NUM * D,), jnp.float32),
        pltpu.SemaphoreType.DMA,
        pltpu.SemaphoreType.DMA,
        pltpu.SemaphoreType.DMA,
        pltpu.SemaphoreType.DMA,
        pltpu.SemaphoreType.DMA,
    ],
)
def _tokenizer(*refs):
    _tokenizer_kernel(*refs)


def kernel(x_num, x_cat, num_weight, num_bias, cat_tables, cat_bias):
    tbl = _prep(*cat_tables, cat_bias)
    xnum_p = jnp.pad(x_num, ((0, 0), (0, L - N_NUM))).reshape(-1)
    out = _tokenizer(
        xnum_p,
        x_cat.reshape(-1),
        tbl,
        num_weight.reshape(-1),
        num_bias.reshape(-1),
    )
    return out.reshape(B, N_TOK, D)


# speedup vs baseline: 1.6654x; 1.6654x over previous
import functools

import jax
import jax.numpy as jnp
from jax import lax
from jax.experimental import pallas as pl
from jax.experimental.pallas import tpu as pltpu
from jax.experimental.pallas import tpu_sc as plsc

D = 64
DP = 128
N_NUM = 13
N_CAT = 26
VOCAB = 1000
B = 16384
N_TOK = N_NUM + N_CAT
ROW = N_TOK * D
CAT0 = N_NUM * D

NC = 2
NS = 16
NW = NC * NS
BPW = B // NW
BC = 8
NCH = BPW // BC
NPAIR = NCH // 2
L = 16
IPC = BC * N_CAT
GR = 104
NG = IPC // GR
CHW = BC * ROW


def _prep_body(*refs):
    t_refs = refs[:N_CAT]
    cb_ref = refs[N_CAT]
    o_ref = refs[N_CAT + 1]
    rows = jnp.concatenate(
        [t_refs[i][...] + cb_ref[i : i + 1, :] for i in range(N_CAT)], axis=0
    )
    o_ref[...] = jnp.concatenate([rows, jnp.zeros_like(rows)], axis=1)


_prep = pl.pallas_call(
    _prep_body,
    grid=(1,),
    in_specs=[pl.BlockSpec((VOCAB, D), lambda i: (0, 0)) for _ in range(N_CAT)]
    + [pl.BlockSpec((N_CAT, D), lambda i: (0, 0))],
    out_specs=pl.BlockSpec((N_CAT * VOCAB, DP), lambda i: (0, 0)),
    out_shape=jax.ShapeDtypeStruct((N_CAT * VOCAB, DP), jnp.float32),
)


def _tokenizer_kernel(xnum_hbm, xcat_hbm, tbl_hbm, w_hbm, nb_hbm, out_hbm,
                      idx_all, xnum_all, cat_a, cat_b, buf_a, buf_b,
                      w_v, nb_v, lsem, gsem_a, gsem_b, wsem_a, wsem_b):
    wid = lax.axis_index("s") * NC + lax.axis_index("c")
    base = wid * BPW

    cps = [
        pltpu.async_copy(w_hbm, w_v, lsem),
        pltpu.async_copy(nb_hbm, nb_v, lsem),
        pltpu.async_copy(xnum_hbm.at[pl.ds(base * L, BPW * L)], xnum_all, lsem),
        pltpu.async_copy(
            xcat_hbm.at[pl.ds(base * N_CAT, BPW * N_CAT)], idx_all, lsem
        ),
    ]
    for cp in cps:
        cp.wait()

    lane = lax.iota(jnp.int32, L)

    def fix_body(k, _):
        sl = pl.ds(k * L, L)
        p = k * L + lane
        off = VOCAB * lax.rem(p, N_CAT)
        idx_all[sl] = jnp.clip(idx_all[sl], 0, VOCAB - 1) + off
        return 0

    lax.fori_loop(0, BPW * N_CAT // L, fix_body, 0)

    def fire_gathers(c, catbuf, sem):
        for g in range(NG):
            pltpu.async_copy(
                tbl_hbm.at[idx_all.at[pl.ds(c * IPC + g * GR, GR)]],
                catbuf.at[pl.ds(g * GR, GR), :],
                sem,
            )

    def drain_gathers(catbuf, sem):
        for g in range(NG):
            pltpu.make_async_copy(
                tbl_hbm.at[idx_all.at[pl.ds(g * GR, GR)]],
                catbuf.at[pl.ds(g * GR, GR), :],
                sem,
            ).wait()

    def fire_write(c, obuf, sem):
        pltpu.async_copy(obuf, out_hbm.at[pl.ds((base + c * BC) * ROW, CHW)], sem)

    def drain_write(obuf, sem):
        pltpu.make_async_copy(obuf, out_hbm.at[pl.ds(0, CHW)], sem).wait()

    def assemble(c, catbuf, obuf):
        @plsc.parallel_loop(0, IPC, unroll=4)
        def cat_move(m):
            b = m // N_CAT
            i = m - b * N_CAT
            dst = b * ROW + CAT0 + i * D
            vals = [catbuf[m, pl.ds(d4 * L, L)] for d4 in range(D // L)]
            for d4 in range(D // L):
                obuf[pl.ds(dst + d4 * L, L)] = vals[d4]

        xvs = [xnum_all[pl.ds((c * BC + b) * L, L)] for b in range(BC)]
        for j in range(N_NUM):
            for d4 in range(D // L):
                wv = w_v[pl.ds(j * D + d4 * L, L)]
                nbv = nb_v[pl.ds(j * D + d4 * L, L)]
                for b in range(BC):
                    obuf[pl.ds(b * ROW + j * D + d4 * L, L)] = xvs[b][j] * wv + nbv

    fire_gathers(0, cat_a, gsem_a)

    def pair_body(k, _):
        c0 = 2 * k
        c1 = 2 * k + 1

        drain_gathers(cat_a, gsem_a)
        fire_gathers(c1, cat_b, gsem_b)

        @pl.when(k > 0)
        def _():
            drain_write(buf_a, wsem_a)

        assemble(c0, cat_a, buf_a)
        fire_write(c0, buf_a, wsem_a)

        drain_gathers(cat_b, gsem_b)

        @pl.when(k < NPAIR - 1)
        def _():
            fire_gathers(c0 + 2, cat_a, gsem_a)

        @pl.when(k > 0)
        def _():
            drain_write(buf_b, wsem_b)

        assemble(c1, cat_b, buf_b)
        fire_write(c1, buf_b, wsem_b)
        return 0

    lax.fori_loop(0, NPAIR, pair_body, 0)
    drain_write(buf_a, wsem_a)
    drain_write(buf_b, wsem_b)


@functools.partial(
    pl.kernel,
    mesh=plsc.VectorSubcoreMesh(core_axis_name="c", subcore_axis_name="s"),
    out_type=jax.ShapeDtypeStruct((B * ROW,), jnp.float32),
    scratch_types=[
        pltpu.VMEM((N_CAT * BPW,), jnp.int32),
        pltpu.VMEM((BPW * L,), jnp.float32),
        pltpu.VMEM((IPC, DP), jnp.float32),
        pltpu.VMEM((IPC, DP), jnp.float32),
        pltpu.VMEM((CHW,), jnp.float32),
        pltpu.VMEM((CHW,), jnp.float32),
        pltpu.VMEM((N_NUM * D,), jnp.float32),
        pltpu.VMEM((N_NUM * D,), jnp.float32),
        pltpu.SemaphoreType.DMA,
        pltpu.SemaphoreType.DMA,
        pltpu.SemaphoreType.DMA,
        pltpu.SemaphoreType.DMA,
        pltpu.SemaphoreType.DMA,
    ],
)
def _tokenizer(*refs):
    _tokenizer_kernel(*refs)


def kernel(x_num, x_cat, num_weight, num_bias, cat_tables, cat_bias):
    tbl = _prep(*[t[:VOCAB] for t in cat_tables], cat_bias)
    xnum_p = jnp.pad(x_num, ((0, 0), (0, L - N_NUM))).reshape(-1)
    out = _tokenizer(
        xnum_p,
        x_cat.reshape(-1),
        tbl,
        num_weight.reshape(-1),
        num_bias.reshape(-1),
    )
    return out.reshape(B, N_TOK, D)


# speedup vs baseline: 1.9388x; 1.1642x over previous
import functools

import jax
import jax.numpy as jnp
from jax import lax
from jax.experimental import pallas as pl
from jax.experimental.pallas import tpu as pltpu
from jax.experimental.pallas import tpu_sc as plsc

D = 64
DP = 128
N_NUM = 13
N_CAT = 26
VOCAB = 1000
B = 16384
N_TOK = N_NUM + N_CAT
ROW = N_TOK * D
CAT0 = N_NUM * D

NC = 2
NS = 16
NW = NC * NS
BPW = B // NW
BC = 4
NCH = BPW // BC
NPAIR = NCH // 2
L = 16
IPC = BC * N_CAT
GR = 104
NG = IPC // GR


def _prep_body(*refs):
    t_refs = refs[:N_CAT]
    cb_ref = refs[N_CAT]
    o_ref = refs[N_CAT + 1]
    rows = jnp.concatenate(
        [t_refs[i][...] + cb_ref[i : i + 1, :] for i in range(N_CAT)], axis=0
    )
    o_ref[...] = jnp.concatenate([rows, jnp.zeros_like(rows)], axis=1)


_prep = pl.pallas_call(
    _prep_body,
    grid=(1,),
    in_specs=[pl.BlockSpec((VOCAB, D), lambda i: (0, 0)) for _ in range(N_CAT)]
    + [pl.BlockSpec((N_CAT, D), lambda i: (0, 0))],
    out_specs=pl.BlockSpec((N_CAT * VOCAB, DP), lambda i: (0, 0)),
    out_shape=jax.ShapeDtypeStruct((N_CAT * VOCAB, DP), jnp.float32),
)


def _tokenizer_kernel(xnum_hbm, xcat_hbm, tbl_hbm, w_hbm, nb_hbm, out_hbm,
                      idx_all, xnum_all, cat_a, cat_b, buf_a, buf_b,
                      w_v, nb_v, lsem, gsem_a, gsem_b, wsem_a, wsem_b):
    wid = lax.axis_index("s") * NC + lax.axis_index("c")
    base = wid * BPW

    cps = [
        pltpu.async_copy(w_hbm, w_v, lsem),
        pltpu.async_copy(nb_hbm, nb_v, lsem),
        pltpu.async_copy(xnum_hbm.at[pl.ds(base * L, BPW * L)], xnum_all, lsem),
        pltpu.async_copy(
            xcat_hbm.at[pl.ds(base * N_CAT, BPW * N_CAT)], idx_all, lsem
        ),
    ]
    for cp in cps:
        cp.wait()

    lane = lax.iota(jnp.int32, L)

    def fix_body(k, _):
        sl = pl.ds(k * L, L)
        p = k * L + lane
        off = VOCAB * lax.rem(p, N_CAT)
        idx_all[sl] = jnp.clip(idx_all[sl], 0, VOCAB - 1) + off
        return 0

    lax.fori_loop(0, BPW * N_CAT // L, fix_body, 0)

    def fire_gathers(c, catbuf, sem):
        for g in range(NG):
            pltpu.async_copy(
                tbl_hbm.at[idx_all.at[pl.ds(c * IPC + g * GR, GR)]],
                catbuf.at[pl.ds(g * GR, GR), :],
                sem,
            )

    def drain_gathers(catbuf, sem):
        for g in range(NG):
            pltpu.make_async_copy(
                tbl_hbm.at[idx_all.at[pl.ds(g * GR, GR)]],
                catbuf.at[pl.ds(g * GR, GR), :],
                sem,
            ).wait()

    def fire_write(c, obuf, sem):
        pltpu.async_copy(obuf, out_hbm.at[pl.ds(base + c * BC, BC)], sem)

    def drain_write(obuf, sem):
        pltpu.make_async_copy(obuf, out_hbm.at[pl.ds(0, BC)], sem).wait()

    def assemble(c, catbuf, obuf):
        @plsc.parallel_loop(0, IPC, unroll=4)
        def cat_move(m):
            b = m // N_CAT
            i = m - b * N_CAT
            vals = [catbuf[m, pl.ds(d4 * L, L)] for d4 in range(D // L)]
            for d4 in range(D // L):
                obuf[b, N_NUM + i, pl.ds(d4 * L, L)] = vals[d4]

        xvs = [xnum_all[pl.ds((c * BC + b) * L, L)] for b in range(BC)]
        for j in range(N_NUM):
            for d4 in range(D // L):
                wv = w_v[pl.ds(j * D + d4 * L, L)]
                nbv = nb_v[pl.ds(j * D + d4 * L, L)]
                for b in range(BC):
                    obuf[b, j, pl.ds(d4 * L, L)] = xvs[b][j] * wv + nbv

    fire_gathers(0, cat_a, gsem_a)

    def pair_body(k, _):
        c0 = 2 * k
        c1 = 2 * k + 1

        drain_gathers(cat_a, gsem_a)
        fire_gathers(c1, cat_b, gsem_b)

        @pl.when(k > 0)
        def _():
            drain_write(buf_a, wsem_a)

        assemble(c0, cat_a, buf_a)
        fire_write(c0, buf_a, wsem_a)

        drain_gathers(cat_b, gsem_b)

        @pl.when(k < NPAIR - 1)
        def _():
            fire_gathers(c0 + 2, cat_a, gsem_a)

        @pl.when(k > 0)
        def _():
            drain_write(buf_b, wsem_b)

        assemble(c1, cat_b, buf_b)
        fire_write(c1, buf_b, wsem_b)
        return 0

    lax.fori_loop(0, NPAIR, pair_body, 0)
    drain_write(buf_a, wsem_a)
    drain_write(buf_b, wsem_b)


@functools.partial(
    pl.kernel,
    mesh=plsc.VectorSubcoreMesh(core_axis_name="c", subcore_axis_name="s"),
    out_type=jax.ShapeDtypeStruct((B, N_TOK, D), jnp.float32),
    scratch_types=[
        pltpu.VMEM((N_CAT * BPW,), jnp.int32),
        pltpu.VMEM((BPW * L,), jnp.float32),
        pltpu.VMEM((IPC, DP), jnp.float32),
        pltpu.VMEM((IPC, DP), jnp.float32),
        pltpu.VMEM((BC, N_TOK, D), jnp.float32),
        pltpu.VMEM((BC, N_TOK, D), jnp.float32),
        pltpu.VMEM((N_NUM * D,), jnp.float32),
        pltpu.VMEM((N_NUM * D,), jnp.float32),
        pltpu.SemaphoreType.DMA,
        pltpu.SemaphoreType.DMA,
        pltpu.SemaphoreType.DMA,
        pltpu.SemaphoreType.DMA,
        pltpu.SemaphoreType.DMA,
    ],
)
def _tokenizer(*refs):
    _tokenizer_kernel(*refs)


def kernel(x_num, x_cat, num_weight, num_bias, cat_tables, cat_bias):
    tbl = _prep(*[t[:VOCAB] for t in cat_tables], cat_bias)
    xnum_p = jnp.pad(x_num, ((0, 0), (0, L - N_NUM))).reshape(-1)
    return _tokenizer(
        xnum_p,
        x_cat.reshape(-1),
        tbl,
        num_weight.reshape(-1),
        num_bias.reshape(-1),
    )
